# baseline (speedup 1.0000x reference)
import functools

import jax
import jax.numpy as jnp
import numpy as np
from jax import lax
from jax.experimental import pallas as pl
from jax.experimental.pallas import tpu as pltpu
from jax.experimental.pallas import tpu_sc as plsc

H = 7
NC, NS = 2, 16
NW = NC * NS
CHUNK = 128
KG = 8
ROW = 8
FOLD = 64
RB = 640


def _sc_gather(ids3, table):
    nchunks = ids3.shape[1]
    per_w = nchunks * CHUNK
    grp = KG * CHUNK
    mesh = plsc.VectorSubcoreMesh(core_axis_name="c", subcore_axis_name="s")

    @functools.partial(
        pl.kernel,
        out_type=jax.ShapeDtypeStruct((NW * per_w, ROW), jnp.float32),
        mesh=mesh,
        scratch_types=[
            pltpu.VMEM((nchunks, CHUNK), jnp.int32),
            pltpu.VMEM((grp, ROW), jnp.float32),
            pltpu.SemaphoreType.DMA,
        ],
        compiler_params=pltpu.CompilerParams(use_tc_tiling_on_sc=False),
    )
    def gather_kernel(ids_hbm, table_hbm, out_hbm, idx_v, rows_v, sem):
        wid = lax.axis_index("s") * NC + lax.axis_index("c")
        pltpu.sync_copy(ids_hbm.at[wid], idx_v)

        def group(g, carry):
            copies = [
                pltpu.async_copy(
                    table_hbm.at[idx_v.at[g * KG + j]],
                    rows_v.at[pl.ds(j * CHUNK, CHUNK)], sem)
                for j in range(KG)
            ]
            for c in copies:
                c.wait()
            pltpu.sync_copy(rows_v, out_hbm.at[pl.ds(wid * per_w + g * grp, grp)])
            return carry

        lax.fori_loop(0, nchunks // KG, group, 0)

    return gather_kernel(ids3, table)


def _patterns(decay_slow, decay_fast):
    h = jnp.arange(1, H + 1, dtype=jnp.float32)
    inv_s = 1.0 / (h ** decay_slow)
    inv_f = 1.0 / (h ** decay_fast)
    r = jnp.arange(ROW * FOLD)[:, None]
    c = jnp.arange(2 * H * FOLD)[None, :]
    j = c // (2 * H)
    k = c % (2 * H)
    slow = k < H
    hk = jnp.where(slow, k, k - H)
    hval = jnp.take(h, hk)
    zero = jnp.float32(0.0)
    mf = (jnp.where((r == ROW * j) & slow, hval, zero)
          + jnp.where((r == ROW * j + 1) & ~slow, hval, zero))
    ma = (jnp.where((r == ROW * j + 2) & slow, jnp.take(inv_s, hk), zero)
          + jnp.where((r == ROW * j + 3) & ~slow, jnp.take(inv_f, hk), zero))
    mp = jnp.where(r == ROW * j + 4, jnp.float32(1.0), zero) + zero * c
    return mf, ma, mp.astype(jnp.float32)


def _tc_expand(g2, mf, ma, mp):
    rows, kdim = g2.shape
    n = mf.shape[1]

    n7 = n // 128

    def body(g_ref, mf_ref, ma_ref, mp_ref, of_ref, oa_ref, op_ref):
        g = g_ref[...]
        for m_ref, o_ref in ((mf_ref, of_ref), (ma_ref, oa_ref), (mp_ref, op_ref)):
            res = jnp.dot(g, m_ref[...], preferred_element_type=jnp.float32)
            o_ref[...] = res.reshape(RB * n7, 128)

    const_spec = pl.BlockSpec((kdim, n), lambda i: (0, 0))
    return pl.pallas_call(
        body,
        grid=(rows // RB,),
        in_specs=[pl.BlockSpec((RB, kdim), lambda i: (i, 0)),
                  const_spec, const_spec, const_spec],
        out_specs=[pl.BlockSpec((RB * n7, 128), lambda i: (i, 0))] * 3,
        out_shape=[jax.ShapeDtypeStruct((rows * n7, 128), jnp.float32)] * 3,
    )(g2, mf, ma, mp)


def kernel(ids, freq_slow, freq_fast, amplitudes, phase, scale_mix,
           decay_slow, decay_fast):
    B, L = ids.shape
    T = B * L
    mix = jax.nn.sigmoid(scale_mix)
    mix_a = mix * amplitudes
    m1_a = (1.0 - mix) * amplitudes
    z = jnp.zeros_like(freq_slow)
    table = jnp.stack(
        [freq_slow, freq_fast, mix_a, m1_a, phase, z, z, z], axis=1)

    nchunks = T // (NW * CHUNK)
    ids3 = ids.reshape(NW, nchunks, CHUNK)
    g = _sc_gather(ids3, table)
    g2 = g.reshape(T // FOLD, ROW * FOLD)

    mf, ma, mp = _patterns(decay_slow, decay_fast)
    of, oa, op = _tc_expand(g2, mf, ma, mp)
    shape = (B, L, 2 * H)
    return of.reshape(shape), oa.reshape(shape), op.reshape(shape)


# speedup vs baseline: 2.8353x; 2.8353x over previous
import functools

import jax
import jax.numpy as jnp
from jax import lax
from jax.experimental import pallas as pl
from jax.experimental.pallas import tpu as pltpu
from jax.experimental.pallas import tpu_sc as plsc

H = 7
NC, NS = 2, 16
NW = NC * NS
CHUNK = 128
KG = 8
ROW = 8
NQ = 5
BB = 512


def _sc_gather(ids3, table):
    nchunks = ids3.shape[1]
    per_w = nchunks * CHUNK
    grp = KG * CHUNK
    total = NW * per_w
    mesh = plsc.VectorSubcoreMesh(core_axis_name="c", subcore_axis_name="s")

    @functools.partial(
        pl.kernel,
        out_type=[jax.ShapeDtypeStruct((total,), jnp.float32)] * NQ,
        mesh=mesh,
        scratch_types=[
            pltpu.VMEM((nchunks, CHUNK), jnp.int32),
            pltpu.VMEM((KG, CHUNK, ROW), jnp.float32),
            pltpu.VMEM((NQ, grp), jnp.float32),
            pltpu.SemaphoreType.DMA,
        ],
        compiler_params=pltpu.CompilerParams(
            use_tc_tiling_on_sc=False, needs_layout_passes=False),
    )
    def gather_kernel(ids_hbm, table_hbm, o0, o1, o2, o3, o4,
                      idx_v, rows_v, st_v, sem):
        wid = lax.axis_index("s") * NC + lax.axis_index("c")
        pltpu.sync_copy(ids_hbm.at[wid], idx_v)
        lane = lax.iota(jnp.int32, 16)
        outs = (o0, o1, o2, o3, o4)

        def group(g, carry):
            copies = [
                pltpu.async_copy(
                    table_hbm.at[idx_v.at[g * KG + j]], rows_v.at[j], sem)
                for j in range(KG)
            ]
            for cp in copies:
                cp.wait()

            def deint(jj, c2):
                jv = jnp.full((16,), jj, jnp.int32)
                for v in range(CHUNK // 16):
                    rowi = lane + (16 * v)
                    for c in range(NQ):
                        cv = jnp.full((16,), c, jnp.int32)
                        val = plsc.load_gather(rows_v, [jv, rowi, cv])
                        st_v[c, pl.ds(jj * CHUNK + 16 * v, 16)] = val
                return c2

            lax.fori_loop(0, KG, deint, 0)
            base = wid * per_w + g * grp
            for c in range(NQ):
                pltpu.sync_copy(st_v.at[c], outs[c].at[pl.ds(base, grp)])
            return carry

        lax.fori_loop(0, nchunks // KG, group, 0)

    return gather_kernel(ids3, table)


def _tc_expand(planes_t, amp14, b, l):
    n = 2 * H

    def body(amp_ref, fs_ref, ff_ref, ma_ref, m1_ref, ph_ref,
             of_ref, oa_ref, op_ref):
        fs = fs_ref[...]
        ff = ff_ref[...]
        ma = ma_ref[...]
        m1 = m1_ref[...]
        ph = ph_ref[...]
        for k in range(n):
            fsrc = fs if k < H else ff
            asrc = ma if k < H else m1
            of_ref[k] = fsrc * jnp.float32((k % H) + 1)
            oa_ref[k] = asrc * amp_ref[0:1, k:k + 1]
            op_ref[k] = ph

    plane_spec = pl.BlockSpec((l, BB), lambda i: (0, i))
    out_spec = pl.BlockSpec((n, l, BB), lambda i: (0, 0, i))
    return pl.pallas_call(
        body,
        grid=(b // BB,),
        in_specs=[pl.BlockSpec((1, n), lambda i: (0, 0))] + [plane_spec] * NQ,
        out_specs=[out_spec] * 3,
        out_shape=[jax.ShapeDtypeStruct((n, l, b), jnp.float32)] * 3,
    )(amp14, *planes_t)


def kernel(ids, freq_slow, freq_fast, amplitudes, phase, scale_mix,
           decay_slow, decay_fast):
    B, L = ids.shape
    T = B * L
    mix = jax.nn.sigmoid(scale_mix)
    mix_a = mix * amplitudes
    m1_a = (1.0 - mix) * amplitudes
    z = jnp.zeros_like(freq_slow)
    table = jnp.stack(
        [freq_slow, freq_fast, mix_a, m1_a, phase, z, z, z], axis=1)

    nchunks = T // (NW * CHUNK)
    ids3 = ids.reshape(NW, nchunks, CHUNK)
    planes = _sc_gather(ids3, table)
    planes_t = [p.reshape(B, L).T for p in planes]

    h = jnp.arange(1, H + 1, dtype=jnp.float32)
    amp14 = jnp.concatenate(
        [1.0 / (h ** decay_slow), 1.0 / (h ** decay_fast)]).reshape(1, 2 * H)
    of, oa, op = _tc_expand(planes_t, amp14, B, L)
    perm = (2, 1, 0)
    return (jnp.transpose(of, perm), jnp.transpose(oa, perm),
            jnp.transpose(op, perm))
